# Initial kernel scaffold; baseline (speedup 1.0000x reference)
import jax
import jax.numpy as jnp
from jax.experimental import pallas as pl


def kernel(x, edge_index, batch, W1, b1, W2, b2, W3, b3, Wm1, bm1, Wm2, bm2):
    raise NotImplementedError("write your pallas kernel here")


# speedup vs baseline: 2.4690x; 2.4690x over previous
import functools

import jax
import jax.numpy as jnp
from jax import lax
from jax.experimental import pallas as pl
from jax.experimental.pallas import tpu as pltpu
from jax.experimental.pallas import tpu_sc as plsc

N_NODES = 10000
N_EDGES = 160000
D = 256
N_GRAPHS = 64
EMB = 10

NC = 2
NS = 16
NT = NC * NS

E_PAD = 163840
E_ROWS = E_PAD // 128
CHUNK_ROWS = 16
PCHUNKS = E_ROWS // CHUNK_ROWS

OWN = 313
ACC_ROWS = 320
SEG = 160256
GR = 128

ROW_BLK = 1000
N_BLK = N_NODES // ROW_BLK

_mesh = plsc.VectorSubcoreMesh(core_axis_name="c", subcore_axis_name="s",
                               num_cores=NC, num_subcores=NS)
_CP = pltpu.CompilerParams(needs_layout_passes=False)


def _tid():
    return lax.axis_index("c") * NS + lax.axis_index("s")


@functools.partial(
    pl.kernel,
    out_type=(jax.ShapeDtypeStruct((NT * SEG,), jnp.int32),
              jax.ShapeDtypeStruct((NT, 128), jnp.int32)),
    mesh=_mesh,
    compiler_params=_CP,
    scratch_types=[
        pltpu.VMEM((CHUNK_ROWS, 128), jnp.int32),
        pltpu.VMEM((CHUNK_ROWS, 128), jnp.int32),
        pltpu.VMEM((CHUNK_ROWS, 128), jnp.int32),
        pltpu.VMEM((CHUNK_ROWS, 128), jnp.int32),
        pltpu.VMEM((128,), jnp.int32),
    ],
)
def _part(src_hbm, dst_hbm, elist_hbm, cnt_hbm, src_b, dst_b, vals, posb, cb):
    t = _tid()
    lo = t * OWN
    hi = lo + OWN
    base = t * SEG
    iota = lax.iota(jnp.int32, 16)

    def chunk_body(k, total):
        row0 = k * CHUNK_ROWS
        pltpu.sync_copy(src_hbm.at[pl.ds(row0, CHUNK_ROWS)], src_b)
        pltpu.sync_copy(dst_hbm.at[pl.ds(row0, CHUNK_ROWS)], dst_b)

        def f_body(i, cnt_vec):
            r = i // 8
            col = (i % 8) * 16
            sv = src_b[r, pl.ds(col, 16)]
            dv = dst_b[r, pl.ds(col, 16)]
            m = (dv >= lo) & (dv < hi)
            loc = cnt_vec + plsc.cumsum(jnp.where(m, 1, 0)) - 1
            pr = loc // 128
            pc = loc - pr * 128
            packed = sv * 512 + (dv - lo)
            plsc.store_scatter(vals, [pr, pc], packed, mask=m)
            plsc.store_scatter(posb, [pr, pc], base + total + loc, mask=m)
            return cnt_vec + jnp.sum(jnp.where(m, 1, 0))

        cnt_vec = lax.fori_loop(0, CHUNK_ROWS * 8, f_body,
                                jnp.zeros((16,), jnp.int32))
        lcnt = cnt_vec[0]
        nch = (lcnt + 127) // 128
        npad = nch * 128
        vb = (lcnt // 16) * 16
        for j in range(9):
            lp = vb + j * 16 + iota
            mpad = (lp >= lcnt) & (lp < npad)
            lpc = jnp.clip(lp, 0, CHUNK_ROWS * 128 - 1)
            pr = lpc // 128
            pc = lpc - pr * 128
            spread = (lp * 37 + t * 131) % N_NODES
            pad_val = spread * 512 + (OWN + lp % (ACC_ROWS - OWN))
            pad_pos = base + N_EDGES + (lp % 128)
            plsc.store_scatter(vals, [pr, pc], pad_val, mask=mpad)
            plsc.store_scatter(posb, [pr, pc], pad_pos, mask=mpad)

        def s_body(j, carry):
            pltpu.sync_copy(vals.at[j], elist_hbm.at[posb.at[j]])
            return carry

        lax.fori_loop(0, nch, s_body, 0)
        return total + lcnt

    total = lax.fori_loop(0, PCHUNKS, chunk_body, jnp.int32(0))

    pad_n = ((total + 127) // 128) * 128 - total
    for j in range(8):
        lp = j * 16 + iota
        spread = (lp * 29 + t * 173) % N_NODES
        pad_val = spread * 512 + (OWN + lp % (ACC_ROWS - OWN))
        pos = jnp.where(lp < pad_n, base + total + lp,
                        base + N_EDGES + 128 + lp)
        plsc.store_scatter(vals, [jnp.zeros((16,), jnp.int32), lp], pad_val)
        plsc.store_scatter(posb, [jnp.zeros((16,), jnp.int32), lp], pos)
    pltpu.sync_copy(vals.at[0], elist_hbm.at[posb.at[0]])

    def cb_body(i, carry):
        cb[pl.ds(i * 16, 16)] = jnp.zeros((16,), jnp.int32) + total
        return carry

    lax.fori_loop(0, 8, cb_body, 0)
    pltpu.sync_copy(cb, cnt_hbm.at[t])


@functools.partial(
    pl.kernel,
    out_type=jax.ShapeDtypeStruct((NT, ACC_ROWS, 16), jnp.float32),
    mesh=_mesh,
    compiler_params=_CP,
    scratch_types=[
        pltpu.VMEM((128,), jnp.int32),
        pltpu.VMEM((ACC_ROWS, 16), jnp.float32),
        pltpu.VMEM((128,), jnp.int32),
    ],
)
def _deg(elist_hbm, cnt_hbm, out_hbm, ebuf, hist, cb):
    t = _tid()
    base = t * SEG
    iota = lax.iota(jnp.int32, 16)
    ones = jnp.ones((16,), jnp.float32)

    def z_body(i, carry):
        hist[i, pl.ds(0, 16)] = jnp.zeros((16,), jnp.float32)
        return carry

    lax.fori_loop(0, ACC_ROWS, z_body, 0)

    pltpu.sync_copy(cnt_hbm.at[t], cb)
    cnt = cb[pl.ds(0, 16)][0]
    nch = (cnt + 127) // 128

    def c_body(j, carry):
        pltpu.sync_copy(elist_hbm.at[pl.ds(base + j * 128, 128)], ebuf)

        def v_body(k, carry2):
            v = ebuf[pl.ds(k * 16, 16)]
            dl = v & 511
            plsc.addupdate_scatter(hist, [dl, iota], ones)
            return carry2

        lax.fori_loop(0, 8, v_body, 0)
        return carry

    lax.fori_loop(0, nch, c_body, 0)
    pltpu.sync_copy(hist, out_hbm.at[t])


@functools.partial(
    pl.kernel,
    out_type=jax.ShapeDtypeStruct((NT, ACC_ROWS, D), jnp.float32),
    mesh=_mesh,
    compiler_params=_CP,
    scratch_types=[
        pltpu.VMEM((GR,), jnp.int32),
        pltpu.VMEM((GR,), jnp.int32),
        pltpu.VMEM((GR,), jnp.int32),
        pltpu.VMEM((GR, D), jnp.float32),
        pltpu.VMEM((ACC_ROWS, D), jnp.float32),
        pltpu.VMEM((128,), jnp.int32),
        pltpu.SemaphoreType.DMA,
    ],
)
def _agg(g_hbm, elist_hbm, cnt_hbm, out_hbm,
         ebuf, gsrc, dlb, rows, acc, cb, sem):
    t = _tid()
    base = t * SEG

    def z_body(i, carry):
        r = i // 16
        acc[r, pl.ds((i % 16) * 16, 16)] = jnp.zeros((16,), jnp.float32)
        return carry

    lax.fori_loop(0, ACC_ROWS * (D // 16), z_body, 0)

    pltpu.sync_copy(cnt_hbm.at[t], cb)
    cnt = cb[pl.ds(0, 16)][0]
    nch = (cnt + GR - 1) // GR

    def c_body(j, carry):
        pltpu.sync_copy(elist_hbm.at[pl.ds(base + j * GR, GR)], ebuf)

        def u_body(k, carry2):
            v = ebuf[pl.ds(k * 16, 16)]
            gsrc[pl.ds(k * 16, 16)] = lax.shift_right_logical(v, 9)
            dlb[pl.ds(k * 16, 16)] = v & 511
            return carry2

        lax.fori_loop(0, GR // 16, u_body, 0)
        pltpu.async_copy(g_hbm.at[gsrc], rows, sem).wait()

        def a_body(k, carry2):
            dlv = dlb[pl.ds(k * 16, 16)]
            for l in range(16):
                d = dlv[l]
                e = k * 16 + l
                for q in range(D // 16):
                    plsc.addupdate(acc.at[d, pl.ds(q * 16, 16)],
                                   rows[e, pl.ds(q * 16, 16)])
            return carry2

        lax.fori_loop(0, GR // 16, a_body, 0)
        return carry

    lax.fori_loop(0, nch, c_body, 0)
    pltpu.sync_copy(acc, out_hbm.at[t])


def _degsum_body(h_ref, o_ref):
    o_ref[...] = jnp.sum(h_ref[...], axis=-1)


def _degsum(hist):
    return pl.pallas_call(
        _degsum_body,
        out_shape=jax.ShapeDtypeStruct((NT, ACC_ROWS), jnp.float32),
    )(hist)


def _t1_body(x_ref, w_ref, deg_ref, g_ref):
    dinv = lax.rsqrt(deg_ref[0] + 1.0)
    h = jnp.dot(x_ref[...], w_ref[...], preferred_element_type=jnp.float32)
    g_ref[...] = h * dinv


def _tmid_body(acc_ref, g_ref, deg_ref, b_ref, w_ref, gn_ref):
    dinv = lax.rsqrt(deg_ref[0] + 1.0)
    h = dinv * (acc_ref[...] + g_ref[...]) + b_ref[...]
    h = jnp.maximum(h, 0.0)
    gn_ref[...] = jnp.dot(h, w_ref[...],
                          preferred_element_type=jnp.float32) * dinv


def _t4_body(acc_ref, g_ref, deg_ref, b_ref, batch_ref,
             wm1_ref, bm1_ref, wm2_ref, bm2_ref, out_ref, pooled):
    i = pl.program_id(0)
    dinv = lax.rsqrt(deg_ref[0] + 1.0)
    h = dinv * (acc_ref[...] + g_ref[...]) + b_ref[...]
    bid = batch_ref[0]
    gi = lax.broadcasted_iota(jnp.int32, (N_GRAPHS, ROW_BLK), 0)
    onehot_t = (gi == bid).astype(jnp.float32)
    contrib = jnp.dot(onehot_t, h.astype(jnp.float32),
                      preferred_element_type=jnp.float32)

    @pl.when(i == 0)
    def _():
        pooled[...] = contrib

    @pl.when(i > 0)
    def _():
        pooled[...] += contrib

    @pl.when(i == pl.num_programs(0) - 1)
    def _():
        z = jnp.tanh(jnp.dot(pooled[...], wm1_ref[...],
                             preferred_element_type=jnp.float32) + bm1_ref[...])
        out_ref[...] = jnp.dot(z, wm2_ref[...],
                               preferred_element_type=jnp.float32) + bm2_ref[...]


def _row_spec():
    return pl.BlockSpec((ROW_BLK, D), lambda i: (i, 0))


def _deg_spec():
    return pl.BlockSpec((1, ROW_BLK, 1), lambda i: (i, 0, 0))


def _full(shape):
    return pl.BlockSpec(shape, lambda i: tuple(0 for _ in shape))


def _t1(x, w1, deg3):
    return pl.pallas_call(
        _t1_body,
        grid=(N_BLK,),
        in_specs=[_row_spec(), _full((D, D)), _deg_spec()],
        out_specs=_row_spec(),
        out_shape=jax.ShapeDtypeStruct((N_NODES, D), jnp.float32),
    )(x, w1, deg3)


def _tmid(acc, g, deg3, b, w_next):
    return pl.pallas_call(
        _tmid_body,
        grid=(N_BLK,),
        in_specs=[_row_spec(), _row_spec(), _deg_spec(), _full((1, D)),
                  _full((D, D))],
        out_specs=_row_spec(),
        out_shape=jax.ShapeDtypeStruct((N_NODES, D), jnp.float32),
    )(acc, g, deg3, b, w_next)


def _t4(acc, g, deg3, b, batch3, wm1, bm1, wm2, bm2):
    return pl.pallas_call(
        _t4_body,
        grid=(N_BLK,),
        in_specs=[_row_spec(), _row_spec(), _deg_spec(), _full((1, D)),
                  pl.BlockSpec((1, 1, ROW_BLK), lambda i: (i, 0, 0)),
                  _full((D, D)), _full((1, D)), _full((D, EMB)),
                  _full((1, EMB))],
        out_specs=_full((N_GRAPHS, EMB)),
        out_shape=jax.ShapeDtypeStruct((N_GRAPHS, EMB), jnp.float32),
        scratch_shapes=[pltpu.VMEM((N_GRAPHS, D), jnp.float32)],
    )(acc, g, deg3, b, batch3, wm1, bm1, wm2, bm2)


def kernel(x, edge_index, batch, W1, b1, W2, b2, W3, b3, Wm1, bm1, Wm2, bm2):
    src = edge_index[0].astype(jnp.int32)
    dst = edge_index[1].astype(jnp.int32)
    npad = E_PAD - N_EDGES
    ar = jnp.arange(npad, dtype=jnp.int32)
    src_p = jnp.concatenate([src, ar % N_NODES]).reshape(E_ROWS, 128)
    dst_p = jnp.concatenate([dst, N_NODES + ar % 128]).reshape(E_ROWS, 128)

    elist, cnts = _part(src_p, dst_p)
    hist = _deg(elist, cnts)
    degs = _degsum(hist)
    degf = degs[:, :OWN].reshape(NT * OWN)[:N_NODES]
    deg3 = degf.reshape(N_BLK, ROW_BLK, 1)
    batch3 = batch.astype(jnp.int32).reshape(N_BLK, 1, ROW_BLK)

    b1r = b1.reshape(1, D)
    b2r = b2.reshape(1, D)
    b3r = b3.reshape(1, D)
    bm1r = bm1.reshape(1, D)
    bm2r = bm2.reshape(1, EMB)

    def agg(g):
        a = _agg(g, elist, cnts)
        return a[:, :OWN].reshape(NT * OWN, D)[:N_NODES]

    g1 = _t1(x, W1, deg3)
    a1 = agg(g1)
    g2 = _tmid(a1, g1, deg3, b1r, W2)
    a2 = agg(g2)
    g3 = _tmid(a2, g2, deg3, b2r, W3)
    a3 = agg(g3)
    return _t4(a3, g3, deg3, b3r, batch3, Wm1, bm1r, Wm2, bm2r)


# speedup vs baseline: 2.4730x; 1.0016x over previous
import functools

import jax
import jax.numpy as jnp
from jax import lax
from jax.experimental import pallas as pl
from jax.experimental.pallas import tpu as pltpu
from jax.experimental.pallas import tpu_sc as plsc

N_NODES = 10000
N_EDGES = 160000
D = 256
N_GRAPHS = 64
EMB = 10

NC = 2
NS = 16
NT = NC * NS

E_PAD = 163840
E_ROWS = E_PAD // 128
CHUNK_ROWS = 16
PCHUNKS = E_ROWS // CHUNK_ROWS

OWN = 313
ACC_ROWS = 320
SEG = 160256
GR = 128

ROW_BLK = 1000
N_BLK = N_NODES // ROW_BLK

_mesh = plsc.VectorSubcoreMesh(core_axis_name="c", subcore_axis_name="s",
                               num_cores=NC, num_subcores=NS)
_CP = pltpu.CompilerParams(needs_layout_passes=False)


def _tid():
    return lax.axis_index("c") * NS + lax.axis_index("s")


@functools.partial(
    pl.kernel,
    out_type=(jax.ShapeDtypeStruct((NT * SEG,), jnp.int32),
              jax.ShapeDtypeStruct((NT, 128), jnp.int32)),
    mesh=_mesh,
    compiler_params=_CP,
    scratch_types=[
        pltpu.VMEM((CHUNK_ROWS, 128), jnp.int32),
        pltpu.VMEM((CHUNK_ROWS, 128), jnp.int32),
        pltpu.VMEM((CHUNK_ROWS, 128), jnp.int32),
        pltpu.VMEM((CHUNK_ROWS, 128), jnp.int32),
        pltpu.VMEM((128,), jnp.int32),
    ],
)
def _part(src_hbm, dst_hbm, elist_hbm, cnt_hbm, src_b, dst_b, vals, posb, cb):
    t = _tid()
    lo = t * OWN
    hi = lo + OWN
    base = t * SEG
    iota = lax.iota(jnp.int32, 16)

    def chunk_body(k, total):
        row0 = k * CHUNK_ROWS
        pltpu.sync_copy(src_hbm.at[pl.ds(row0, CHUNK_ROWS)], src_b)
        pltpu.sync_copy(dst_hbm.at[pl.ds(row0, CHUNK_ROWS)], dst_b)
        btv = jnp.zeros((16,), jnp.int32) + (base + total)

        def row_body(r, cnt_vec):
            for cidx in range(8):
                sv = src_b[r, pl.ds(cidx * 16, 16)]
                dv = dst_b[r, pl.ds(cidx * 16, 16)]
                m = (dv >= lo) & (dv < hi)
                cum = plsc.cumsum(jnp.where(m, 1, 0))
                loc = cnt_vec + cum - 1
                pr = lax.shift_right_arithmetic(loc, 7)
                pc = loc & 127
                packed = sv * 512 + (dv - lo)
                plsc.store_scatter(vals, [pr, pc], packed, mask=m)
                plsc.store_scatter(posb, [pr, pc], btv + loc, mask=m)
                cnt_vec = cnt_vec + cum[15]
            return cnt_vec

        cnt_vec = lax.fori_loop(0, CHUNK_ROWS, row_body,
                                jnp.zeros((16,), jnp.int32))
        lcnt = cnt_vec[0]
        nch = (lcnt + 127) >> 7
        npad = nch * 128
        vb = (lcnt >> 4) << 4
        for j in range(9):
            lp = vb + j * 16 + iota
            mpad = (lp >= lcnt) & (lp < npad)
            lpc = jnp.clip(lp, 0, CHUNK_ROWS * 128 - 1)
            pr = lax.shift_right_arithmetic(lpc, 7)
            pc = lpc & 127
            spread = (lp * 37 + t * 131) % N_NODES
            pad_val = spread * 512 + (OWN + lp % (ACC_ROWS - OWN))
            pad_pos = base + N_EDGES + (lp & 127)
            plsc.store_scatter(vals, [pr, pc], pad_val, mask=mpad)
            plsc.store_scatter(posb, [pr, pc], pad_pos, mask=mpad)

        def s_body(j, carry):
            pltpu.sync_copy(vals.at[j], elist_hbm.at[posb.at[j]])
            return carry

        lax.fori_loop(0, nch, s_body, 0)
        return total + lcnt

    total = lax.fori_loop(0, PCHUNKS, chunk_body, jnp.int32(0))

    pad_n = ((total + 127) // 128) * 128 - total
    for j in range(8):
        lp = j * 16 + iota
        spread = (lp * 29 + t * 173) % N_NODES
        pad_val = spread * 512 + (OWN + lp % (ACC_ROWS - OWN))
        pos = jnp.where(lp < pad_n, base + total + lp,
                        base + N_EDGES + 128 + lp)
        plsc.store_scatter(vals, [jnp.zeros((16,), jnp.int32), lp], pad_val)
        plsc.store_scatter(posb, [jnp.zeros((16,), jnp.int32), lp], pos)
    pltpu.sync_copy(vals.at[0], elist_hbm.at[posb.at[0]])

    def cb_body(i, carry):
        cb[pl.ds(i * 16, 16)] = jnp.zeros((16,), jnp.int32) + total
        return carry

    lax.fori_loop(0, 8, cb_body, 0)
    pltpu.sync_copy(cb, cnt_hbm.at[t])


@functools.partial(
    pl.kernel,
    out_type=jax.ShapeDtypeStruct((NT, ACC_ROWS, 16), jnp.float32),
    mesh=_mesh,
    compiler_params=_CP,
    scratch_types=[
        pltpu.VMEM((128,), jnp.int32),
        pltpu.VMEM((ACC_ROWS, 16), jnp.float32),
        pltpu.VMEM((128,), jnp.int32),
    ],
)
def _deg(elist_hbm, cnt_hbm, out_hbm, ebuf, hist, cb):
    t = _tid()
    base = t * SEG
    iota = lax.iota(jnp.int32, 16)
    ones = jnp.ones((16,), jnp.float32)

    def z_body(i, carry):
        hist[i, pl.ds(0, 16)] = jnp.zeros((16,), jnp.float32)
        return carry

    lax.fori_loop(0, ACC_ROWS, z_body, 0)

    pltpu.sync_copy(cnt_hbm.at[t], cb)
    cnt = cb[pl.ds(0, 16)][0]
    nch = (cnt + 127) // 128

    def c_body(j, carry):
        pltpu.sync_copy(elist_hbm.at[pl.ds(base + j * 128, 128)], ebuf)

        def v_body(k, carry2):
            v = ebuf[pl.ds(k * 16, 16)]
            dl = v & 511
            plsc.addupdate_scatter(hist, [dl, iota], ones)
            return carry2

        lax.fori_loop(0, 8, v_body, 0)
        return carry

    lax.fori_loop(0, nch, c_body, 0)
    pltpu.sync_copy(hist, out_hbm.at[t])


@functools.partial(
    pl.kernel,
    out_type=jax.ShapeDtypeStruct((NT, ACC_ROWS, D), jnp.float32),
    mesh=_mesh,
    compiler_params=_CP,
    scratch_types=[
        pltpu.VMEM((GR,), jnp.int32),
        pltpu.VMEM((GR,), jnp.int32),
        pltpu.VMEM((GR,), jnp.int32),
        pltpu.VMEM((GR, D), jnp.float32),
        pltpu.VMEM((ACC_ROWS, D), jnp.float32),
        pltpu.VMEM((128,), jnp.int32),
        pltpu.SemaphoreType.DMA,
    ],
)
def _agg(g_hbm, elist_hbm, cnt_hbm, out_hbm,
         ebuf, gsrc, dlb, rows, acc, cb, sem):
    t = _tid()
    base = t * SEG

    def z_body(i, carry):
        r = i // 16
        acc[r, pl.ds((i % 16) * 16, 16)] = jnp.zeros((16,), jnp.float32)
        return carry

    lax.fori_loop(0, ACC_ROWS * (D // 16), z_body, 0)

    pltpu.sync_copy(cnt_hbm.at[t], cb)
    cnt = cb[pl.ds(0, 16)][0]
    nch = (cnt + GR - 1) // GR

    def c_body(j, carry):
        pltpu.sync_copy(elist_hbm.at[pl.ds(base + j * GR, GR)], ebuf)

        for k in range(GR // 16):
            v = ebuf[pl.ds(k * 16, 16)]
            gsrc[pl.ds(k * 16, 16)] = lax.shift_right_logical(v, 9)
            dlb[pl.ds(k * 16, 16)] = v & 511
        pltpu.async_copy(g_hbm.at[gsrc], rows, sem).wait()

        def a_body(k, carry2):
            dlv = dlb[pl.ds(k * 16, 16)]
            for l in range(16):
                d = dlv[l]
                e = k * 16 + l
                for q in range(D // 16):
                    plsc.addupdate(acc.at[d, pl.ds(q * 16, 16)],
                                   rows[e, pl.ds(q * 16, 16)])
            return carry2

        lax.fori_loop(0, GR // 16, a_body, 0)
        return carry

    lax.fori_loop(0, nch, c_body, 0)
    pltpu.sync_copy(acc, out_hbm.at[t])


def _degsum_body(h_ref, o_ref):
    o_ref[...] = jnp.sum(h_ref[...], axis=-1)


def _degsum(hist):
    return pl.pallas_call(
        _degsum_body,
        out_shape=jax.ShapeDtypeStruct((NT, ACC_ROWS), jnp.float32),
    )(hist)


def _t1_body(x_ref, w_ref, deg_ref, g_ref):
    dinv = lax.rsqrt(deg_ref[0] + 1.0)
    h = jnp.dot(x_ref[...], w_ref[...], preferred_element_type=jnp.float32)
    g_ref[...] = h * dinv


def _tmid_body(acc_ref, g_ref, deg_ref, b_ref, w_ref, gn_ref):
    dinv = lax.rsqrt(deg_ref[0] + 1.0)
    h = dinv * (acc_ref[...] + g_ref[...]) + b_ref[...]
    h = jnp.maximum(h, 0.0)
    gn_ref[...] = jnp.dot(h, w_ref[...],
                          preferred_element_type=jnp.float32) * dinv


def _t4_body(acc_ref, g_ref, deg_ref, b_ref, batch_ref,
             wm1_ref, bm1_ref, wm2_ref, bm2_ref, out_ref, pooled):
    i = pl.program_id(0)
    dinv = lax.rsqrt(deg_ref[0] + 1.0)
    h = dinv * (acc_ref[...] + g_ref[...]) + b_ref[...]
    bid = batch_ref[0]
    gi = lax.broadcasted_iota(jnp.int32, (N_GRAPHS, ROW_BLK), 0)
    onehot_t = (gi == bid).astype(jnp.float32)
    contrib = jnp.dot(onehot_t, h.astype(jnp.float32),
                      preferred_element_type=jnp.float32)

    @pl.when(i == 0)
    def _():
        pooled[...] = contrib

    @pl.when(i > 0)
    def _():
        pooled[...] += contrib

    @pl.when(i == pl.num_programs(0) - 1)
    def _():
        z = jnp.tanh(jnp.dot(pooled[...], wm1_ref[...],
                             preferred_element_type=jnp.float32) + bm1_ref[...])
        out_ref[...] = jnp.dot(z, wm2_ref[...],
                               preferred_element_type=jnp.float32) + bm2_ref[...]


def _row_spec():
    return pl.BlockSpec((ROW_BLK, D), lambda i: (i, 0))


def _deg_spec():
    return pl.BlockSpec((1, ROW_BLK, 1), lambda i: (i, 0, 0))


def _full(shape):
    return pl.BlockSpec(shape, lambda i: tuple(0 for _ in shape))


def _t1(x, w1, deg3):
    return pl.pallas_call(
        _t1_body,
        grid=(N_BLK,),
        in_specs=[_row_spec(), _full((D, D)), _deg_spec()],
        out_specs=_row_spec(),
        out_shape=jax.ShapeDtypeStruct((N_NODES, D), jnp.float32),
    )(x, w1, deg3)


def _tmid(acc, g, deg3, b, w_next):
    return pl.pallas_call(
        _tmid_body,
        grid=(N_BLK,),
        in_specs=[_row_spec(), _row_spec(), _deg_spec(), _full((1, D)),
                  _full((D, D))],
        out_specs=_row_spec(),
        out_shape=jax.ShapeDtypeStruct((N_NODES, D), jnp.float32),
    )(acc, g, deg3, b, w_next)


def _t4(acc, g, deg3, b, batch3, wm1, bm1, wm2, bm2):
    return pl.pallas_call(
        _t4_body,
        grid=(N_BLK,),
        in_specs=[_row_spec(), _row_spec(), _deg_spec(), _full((1, D)),
                  pl.BlockSpec((1, 1, ROW_BLK), lambda i: (i, 0, 0)),
                  _full((D, D)), _full((1, D)), _full((D, EMB)),
                  _full((1, EMB))],
        out_specs=_full((N_GRAPHS, EMB)),
        out_shape=jax.ShapeDtypeStruct((N_GRAPHS, EMB), jnp.float32),
        scratch_shapes=[pltpu.VMEM((N_GRAPHS, D), jnp.float32)],
    )(acc, g, deg3, b, batch3, wm1, bm1, wm2, bm2)


def kernel(x, edge_index, batch, W1, b1, W2, b2, W3, b3, Wm1, bm1, Wm2, bm2):
    src = edge_index[0].astype(jnp.int32)
    dst = edge_index[1].astype(jnp.int32)
    npad = E_PAD - N_EDGES
    ar = jnp.arange(npad, dtype=jnp.int32)
    src_p = jnp.concatenate([src, ar % N_NODES]).reshape(E_ROWS, 128)
    dst_p = jnp.concatenate([dst, N_NODES + ar % 128]).reshape(E_ROWS, 128)

    elist, cnts = _part(src_p, dst_p)
    hist = _deg(elist, cnts)
    degs = _degsum(hist)
    degf = degs[:, :OWN].reshape(NT * OWN)[:N_NODES]
    deg3 = degf.reshape(N_BLK, ROW_BLK, 1)
    batch3 = batch.astype(jnp.int32).reshape(N_BLK, 1, ROW_BLK)

    b1r = b1.reshape(1, D)
    b2r = b2.reshape(1, D)
    b3r = b3.reshape(1, D)
    bm1r = bm1.reshape(1, D)
    bm2r = bm2.reshape(1, EMB)

    def agg(g):
        a = _agg(g, elist, cnts)
        return a[:, :OWN].reshape(NT * OWN, D)[:N_NODES]

    g1 = _t1(x, W1, deg3)
    a1 = agg(g1)
    g2 = _tmid(a1, g1, deg3, b1r, W2)
    a2 = agg(g2)
    g3 = _tmid(a2, g2, deg3, b2r, W3)
    a3 = agg(g3)
    return _t4(a3, g3, deg3, b3r, batch3, Wm1, bm1r, Wm2, bm2r)


# speedup vs baseline: 4.6988x; 1.9001x over previous
import functools

import jax
import jax.numpy as jnp
from jax import lax
from jax.experimental import pallas as pl
from jax.experimental.pallas import tpu as pltpu
from jax.experimental.pallas import tpu_sc as plsc

N_NODES = 10000
N_EDGES = 160000
D = 256
N_GRAPHS = 64
EMB = 10

NC = 2
NS = 16
NT = NC * NS

E_PAD = 163840
E_ROWS = E_PAD // 128
CHUNK_ROWS = 16
PCHUNKS = E_ROWS // CHUNK_ROWS

OWN = 313
ACC_ROWS = 320
SEG_ROWS = 1264
GR = 128

ROW_BLK = 1000
N_BLK = N_NODES // ROW_BLK

_mesh = plsc.VectorSubcoreMesh(core_axis_name="c", subcore_axis_name="s",
                               num_cores=NC, num_subcores=NS)
_CP = pltpu.CompilerParams(needs_layout_passes=False)


def _tid():
    return lax.axis_index("c") * NS + lax.axis_index("s")


@functools.partial(
    pl.kernel,
    out_type=(jax.ShapeDtypeStruct((NT * SEG_ROWS, 128), jnp.int32),
              jax.ShapeDtypeStruct((NT, 128), jnp.int32)),
    mesh=_mesh,
    compiler_params=_CP,
    scratch_types=[
        pltpu.VMEM((CHUNK_ROWS, 128), jnp.int32),
        pltpu.VMEM((CHUNK_ROWS, 128), jnp.int32),
        pltpu.VMEM((32, 128), jnp.int32),
        pltpu.VMEM((1, 16), jnp.int32),
        pltpu.VMEM((128,), jnp.int32),
    ],
)
def _part(src_hbm, dst_hbm, elist_hbm, cnt_hbm, src_b, dst_b, vals, idxb, cb):
    t = _tid()
    lo = t * OWN
    hi = lo + OWN
    rbase = t * SEG_ROWS
    iota = lax.iota(jnp.int32, 16)
    z16 = jnp.zeros((16,), jnp.int32)

    def flush(fl):
        plsc.store_scatter(idxb, [z16, iota], rbase + fl + iota)

        @pl.when((fl & 31) == 0)
        def _():
            pltpu.sync_copy(vals.at[pl.ds(0, 16)], elist_hbm.at[idxb.at[0]])

        @pl.when((fl & 31) != 0)
        def _():
            pltpu.sync_copy(vals.at[pl.ds(16, 16)], elist_hbm.at[idxb.at[0]])

    def chunk_body(k, carry):
        total, fl = carry
        row0 = k * CHUNK_ROWS
        pltpu.sync_copy(src_hbm.at[pl.ds(row0, CHUNK_ROWS)], src_b)
        pltpu.sync_copy(dst_hbm.at[pl.ds(row0, CHUNK_ROWS)], dst_b)

        def row_body(r, cnt_vec):
            for cidx in range(8):
                sv = src_b[r, pl.ds(cidx * 16, 16)]
                dv = dst_b[r, pl.ds(cidx * 16, 16)]
                m = (dv >= lo) & (dv < hi)
                cum = plsc.cumsum(jnp.where(m, 1, 0))
                loc = cnt_vec + cum - 1
                pr = lax.shift_right_arithmetic(loc, 7) & 31
                pc = loc & 127
                packed = sv * 512 + (dv - lo)
                plsc.store_scatter(vals, [pr, pc], packed, mask=m)
                cnt_vec = cnt_vec + cum[15]
            return cnt_vec

        cnt_vec = lax.fori_loop(0, CHUNK_ROWS, row_body, z16 + total)
        total = cnt_vec[0]

        @pl.when(total - fl * 128 >= 2048)
        def _():
            flush(fl)

        fl = jnp.where(total - fl * 128 >= 2048, fl + 16, fl)
        return total, fl

    total, fl = lax.fori_loop(0, PCHUNKS, chunk_body,
                              (jnp.int32(0), jnp.int32(0)))

    npad = ((total + 127) >> 7) << 7
    vb = (total >> 4) << 4
    for j in range(9):
        lp = vb + j * 16 + iota
        mpad = (lp >= total) & (lp < npad)
        pr = lax.shift_right_arithmetic(lp, 7) & 31
        pc = lp & 127
        spread = (lp * 37 + t * 131) % N_NODES
        pad_val = spread * 512 + (OWN + (lp & 7) % (ACC_ROWS - OWN))
        plsc.store_scatter(vals, [pr, pc], pad_val, mask=mpad)

    @pl.when(total - fl * 128 > 0)
    def _():
        flush(fl)

    def cb_body(i, carry):
        cb[pl.ds(i * 16, 16)] = z16 + total
        return carry

    lax.fori_loop(0, 8, cb_body, 0)
    pltpu.sync_copy(cb, cnt_hbm.at[t])


@functools.partial(
    pl.kernel,
    out_type=jax.ShapeDtypeStruct((NT, ACC_ROWS, 16), jnp.float32),
    mesh=_mesh,
    compiler_params=_CP,
    scratch_types=[
        pltpu.VMEM((128,), jnp.int32),
        pltpu.VMEM((ACC_ROWS, 16), jnp.float32),
        pltpu.VMEM((128,), jnp.int32),
    ],
)
def _deg(elist_hbm, cnt_hbm, out_hbm, ebuf, hist, cb):
    t = _tid()
    base = t * SEG_ROWS
    iota = lax.iota(jnp.int32, 16)
    ones = jnp.ones((16,), jnp.float32)

    def z_body(i, carry):
        hist[i, pl.ds(0, 16)] = jnp.zeros((16,), jnp.float32)
        return carry

    lax.fori_loop(0, ACC_ROWS, z_body, 0)

    pltpu.sync_copy(cnt_hbm.at[t], cb)
    cnt = cb[pl.ds(0, 16)][0]
    nch = (cnt + 127) // 128

    def c_body(j, carry):
        pltpu.sync_copy(elist_hbm.at[base + j], ebuf)

        def v_body(k, carry2):
            v = ebuf[pl.ds(k * 16, 16)]
            dl = v & 511
            plsc.addupdate_scatter(hist, [dl, iota], ones)
            return carry2

        lax.fori_loop(0, 8, v_body, 0)
        return carry

    lax.fori_loop(0, nch, c_body, 0)
    pltpu.sync_copy(hist, out_hbm.at[t])


@functools.partial(
    pl.kernel,
    out_type=jax.ShapeDtypeStruct((NT, ACC_ROWS, D), jnp.float32),
    mesh=_mesh,
    compiler_params=_CP,
    scratch_types=[
        pltpu.VMEM((GR,), jnp.int32),
        pltpu.VMEM((GR,), jnp.int32),
        pltpu.VMEM((GR,), jnp.int32),
        pltpu.VMEM((GR, D), jnp.float32),
        pltpu.VMEM((ACC_ROWS, D), jnp.float32),
        pltpu.VMEM((128,), jnp.int32),
        pltpu.SemaphoreType.DMA,
    ],
)
def _agg(g_hbm, elist_hbm, cnt_hbm, out_hbm,
         ebuf, gsrc, dlb, rows, acc, cb, sem):
    t = _tid()
    base = t * SEG_ROWS

    def z_body(i, carry):
        r = i // 16
        acc[r, pl.ds((i % 16) * 16, 16)] = jnp.zeros((16,), jnp.float32)
        return carry

    lax.fori_loop(0, ACC_ROWS * (D // 16), z_body, 0)

    pltpu.sync_copy(cnt_hbm.at[t], cb)
    cnt = cb[pl.ds(0, 16)][0]
    nch = (cnt + GR - 1) // GR

    def c_body(j, carry):
        pltpu.sync_copy(elist_hbm.at[base + j], ebuf)

        for k in range(GR // 16):
            v = ebuf[pl.ds(k * 16, 16)]
            gsrc[pl.ds(k * 16, 16)] = lax.shift_right_logical(v, 9)
            dlb[pl.ds(k * 16, 16)] = v & 511
        pltpu.async_copy(g_hbm.at[gsrc], rows, sem).wait()

        def a_body(k, carry2):
            dlv = dlb[pl.ds(k * 16, 16)]
            for l in range(16):
                d = dlv[l]
                e = k * 16 + l
                for q in range(D // 16):
                    plsc.addupdate(acc.at[d, pl.ds(q * 16, 16)],
                                   rows[e, pl.ds(q * 16, 16)])
            return carry2

        lax.fori_loop(0, GR // 16, a_body, 0)
        return carry

    lax.fori_loop(0, nch, c_body, 0)
    pltpu.sync_copy(acc, out_hbm.at[t])


def _degsum_body(h_ref, o_ref):
    o_ref[...] = jnp.sum(h_ref[...], axis=-1)


def _degsum(hist):
    return pl.pallas_call(
        _degsum_body,
        out_shape=jax.ShapeDtypeStruct((NT, ACC_ROWS), jnp.float32),
    )(hist)


def _t1_body(x_ref, w_ref, deg_ref, g_ref):
    dinv = lax.rsqrt(deg_ref[0] + 1.0)
    h = jnp.dot(x_ref[...], w_ref[...], preferred_element_type=jnp.float32)
    g_ref[...] = h * dinv


def _tmid_body(acc_ref, g_ref, deg_ref, b_ref, w_ref, gn_ref):
    dinv = lax.rsqrt(deg_ref[0] + 1.0)
    h = dinv * (acc_ref[...] + g_ref[...]) + b_ref[...]
    h = jnp.maximum(h, 0.0)
    gn_ref[...] = jnp.dot(h, w_ref[...],
                          preferred_element_type=jnp.float32) * dinv


def _t4_body(acc_ref, g_ref, deg_ref, b_ref, batch_ref,
             wm1_ref, bm1_ref, wm2_ref, bm2_ref, out_ref, pooled):
    i = pl.program_id(0)
    dinv = lax.rsqrt(deg_ref[0] + 1.0)
    h = dinv * (acc_ref[...] + g_ref[...]) + b_ref[...]
    bid = batch_ref[0]
    gi = lax.broadcasted_iota(jnp.int32, (N_GRAPHS, ROW_BLK), 0)
    onehot_t = (gi == bid).astype(jnp.float32)
    contrib = jnp.dot(onehot_t, h.astype(jnp.float32),
                      preferred_element_type=jnp.float32)

    @pl.when(i == 0)
    def _():
        pooled[...] = contrib

    @pl.when(i > 0)
    def _():
        pooled[...] += contrib

    @pl.when(i == pl.num_programs(0) - 1)
    def _():
        z = jnp.tanh(jnp.dot(pooled[...], wm1_ref[...],
                             preferred_element_type=jnp.float32) + bm1_ref[...])
        out_ref[...] = jnp.dot(z, wm2_ref[...],
                               preferred_element_type=jnp.float32) + bm2_ref[...]


def _row_spec():
    return pl.BlockSpec((ROW_BLK, D), lambda i: (i, 0))


def _deg_spec():
    return pl.BlockSpec((1, ROW_BLK, 1), lambda i: (i, 0, 0))


def _full(shape):
    return pl.BlockSpec(shape, lambda i: tuple(0 for _ in shape))


def _t1(x, w1, deg3):
    return pl.pallas_call(
        _t1_body,
        grid=(N_BLK,),
        in_specs=[_row_spec(), _full((D, D)), _deg_spec()],
        out_specs=_row_spec(),
        out_shape=jax.ShapeDtypeStruct((N_NODES, D), jnp.float32),
    )(x, w1, deg3)


def _tmid(acc, g, deg3, b, w_next):
    return pl.pallas_call(
        _tmid_body,
        grid=(N_BLK,),
        in_specs=[_row_spec(), _row_spec(), _deg_spec(), _full((1, D)),
                  _full((D, D))],
        out_specs=_row_spec(),
        out_shape=jax.ShapeDtypeStruct((N_NODES, D), jnp.float32),
    )(acc, g, deg3, b, w_next)


def _t4(acc, g, deg3, b, batch3, wm1, bm1, wm2, bm2):
    return pl.pallas_call(
        _t4_body,
        grid=(N_BLK,),
        in_specs=[_row_spec(), _row_spec(), _deg_spec(), _full((1, D)),
                  pl.BlockSpec((1, 1, ROW_BLK), lambda i: (i, 0, 0)),
                  _full((D, D)), _full((1, D)), _full((D, EMB)),
                  _full((1, EMB))],
        out_specs=_full((N_GRAPHS, EMB)),
        out_shape=jax.ShapeDtypeStruct((N_GRAPHS, EMB), jnp.float32),
        scratch_shapes=[pltpu.VMEM((N_GRAPHS, D), jnp.float32)],
    )(acc, g, deg3, b, batch3, wm1, bm1, wm2, bm2)


def kernel(x, edge_index, batch, W1, b1, W2, b2, W3, b3, Wm1, bm1, Wm2, bm2):
    src = edge_index[0].astype(jnp.int32)
    dst = edge_index[1].astype(jnp.int32)
    npad = E_PAD - N_EDGES
    ar = jnp.arange(npad, dtype=jnp.int32)
    src_p = jnp.concatenate([src, ar % N_NODES]).reshape(E_ROWS, 128)
    dst_p = jnp.concatenate([dst, N_NODES + ar % 128]).reshape(E_ROWS, 128)

    elist, cnts = _part(src_p, dst_p)
    hist = _deg(elist, cnts)
    degs = _degsum(hist)
    degf = degs[:, :OWN].reshape(NT * OWN)[:N_NODES]
    deg3 = degf.reshape(N_BLK, ROW_BLK, 1)
    batch3 = batch.astype(jnp.int32).reshape(N_BLK, 1, ROW_BLK)

    b1r = b1.reshape(1, D)
    b2r = b2.reshape(1, D)
    b3r = b3.reshape(1, D)
    bm1r = bm1.reshape(1, D)
    bm2r = bm2.reshape(1, EMB)

    def agg(g):
        a = _agg(g, elist, cnts)
        return a[:, :OWN].reshape(NT * OWN, D)[:N_NODES]

    g1 = _t1(x, W1, deg3)
    a1 = agg(g1)
    g2 = _tmid(a1, g1, deg3, b1r, W2)
    a2 = agg(g2)
    g3 = _tmid(a2, g2, deg3, b2r, W3)
    a3 = agg(g3)
    return _t4(a3, g3, deg3, b3r, batch3, Wm1, bm1r, Wm2, bm2r)
